# baseline (speedup 1.0000x reference)
import functools

import jax
import jax.numpy as jnp
from jax import lax
from jax.experimental import pallas as pl
from jax.experimental.pallas import tpu as pltpu
from jax.experimental.pallas import tpu_sc as plsc

N_NODES = 10000
N_EDGES = 320000
IN_CH = 128
HID = 256
HALF = 128
QTR = 64
NUM_LAYERS = 3

NUM_TILES = 16
CHUNK = 128
CHUNKS_PER_TILE = 159
E_PAD = NUM_TILES * CHUNKS_PER_TILE * CHUNK
N_DUMMY = 240
AGG_ROWS = 10240
ROWS_PER_SUB = AGG_ROWS // NUM_TILES

_MESH = plsc.VectorSubcoreMesh(core_axis_name="c", subcore_axis_name="s")


def _agg_body(h4, gsrc, gdst, zrow, out, gsrc_v, gdst_v, rows_a, rows_b,
              rows_c, agg_sh, gsem_a, gsem_b, gsem_c):
    c = lax.axis_index("c")
    s = lax.axis_index("s")
    pltpu.sync_copy(gdst.at[s], gdst_v)

    def gather(j, buf, sem):
        pltpu.async_copy(h4.at[gsrc_v.at[j]], buf, sem)

    def gwait(buf, sem):
        pltpu.make_async_copy(h4, buf, sem).wait()

    def scatter(j, buf):
        pltpu.sync_copy(buf, agg_sh.at[gdst_v.at[j]], add=True)

    for p in range(2):
        pltpu.sync_copy(gsrc.at[c, p, s], gsrc_v)
        pltpu.sync_copy(zrow, rows_a)
        for t in range(ROWS_PER_SUB // CHUNK):
            pltpu.sync_copy(
                rows_a, agg_sh.at[pl.ds(s * ROWS_PER_SUB + t * CHUNK, CHUNK)]
            )
        plsc.subcore_barrier()

        bufs = [(rows_a, gsem_a), (rows_b, gsem_b), (rows_c, gsem_c)]
        gather(0, *bufs[0])
        gather(1, *bufs[1])

        def tri_step(i, carry):
            j = 3 * i
            for k in range(3):
                buf, sem = bufs[k]
                nbuf, nsem = bufs[(k + 2) % 3]

                @pl.when(j + k + 2 < CHUNKS_PER_TILE)
                def _():
                    gather(j + k + 2, nbuf, nsem)

                gwait(buf, sem)
                scatter(j + k, buf)
            return carry

        lax.fori_loop(0, CHUNKS_PER_TILE // 3, tri_step, 0)
        plsc.subcore_barrier()
        pltpu.sync_copy(
            agg_sh.at[pl.ds(s * ROWS_PER_SUB, ROWS_PER_SUB)],
            out.at[c, p, pl.ds(s * ROWS_PER_SUB, ROWS_PER_SUB)],
        )


_agg_call = pl.kernel(
    _agg_body,
    out_type=jax.ShapeDtypeStruct((2, 2, AGG_ROWS, QTR), jnp.float32),
    mesh=_MESH,
    compiler_params=pltpu.CompilerParams(use_tc_tiling_on_sc=False),
    scratch_types=[
        pltpu.VMEM((CHUNKS_PER_TILE, CHUNK), jnp.int32),
        pltpu.VMEM((CHUNKS_PER_TILE, CHUNK), jnp.int32),
        pltpu.VMEM((CHUNK, QTR), jnp.float32),
        pltpu.VMEM((CHUNK, QTR), jnp.float32),
        pltpu.VMEM((CHUNK, QTR), jnp.float32),
        pltpu.VMEM_SHARED((AGG_ROWS, QTR), jnp.float32),
        pltpu.SemaphoreType.DMA,
        pltpu.SemaphoreType.DMA,
        pltpu.SemaphoreType.DMA,
    ],
)


def _deg_body(gdst, ones_v_src, z_v_src, out, gdst_v, ones_v, z_v, deg_sh):
    c = lax.axis_index("c")
    s = lax.axis_index("s")
    pltpu.sync_copy(gdst.at[s], gdst_v)
    pltpu.sync_copy(ones_v_src, ones_v)
    pltpu.sync_copy(z_v_src, z_v)
    for t in range(ROWS_PER_SUB // CHUNK):
        pltpu.sync_copy(z_v, deg_sh.at[pl.ds(s * ROWS_PER_SUB + t * CHUNK, CHUNK)])
    plsc.subcore_barrier()

    def chunk_step(j, carry):
        pltpu.sync_copy(ones_v, deg_sh.at[gdst_v.at[j]], add=True)
        return carry

    lax.fori_loop(0, CHUNKS_PER_TILE, chunk_step, 0)
    plsc.subcore_barrier()
    pltpu.sync_copy(
        deg_sh.at[pl.ds(s * ROWS_PER_SUB, ROWS_PER_SUB)],
        out.at[c, pl.ds(s * ROWS_PER_SUB, ROWS_PER_SUB)],
    )


_deg_call = pl.kernel(
    _deg_body,
    out_type=jax.ShapeDtypeStruct((2, AGG_ROWS, 16), jnp.float32),
    mesh=_MESH,
    compiler_params=pltpu.CompilerParams(use_tc_tiling_on_sc=False),
    scratch_types=[
        pltpu.VMEM((CHUNKS_PER_TILE, CHUNK), jnp.int32),
        pltpu.VMEM((CHUNK, 16), jnp.float32),
        pltpu.VMEM((CHUNK, 16), jnp.float32),
        pltpu.VMEM_SHARED((AGG_ROWS, 16), jnp.float32),
    ],
)


_BLK = 2000


def _mlp_body(x_ref, w1_ref, b1_ref, w2_ref, b2_ref, o_ref):
    h = jnp.dot(x_ref[...], w1_ref[...], preferred_element_type=jnp.float32)
    h = jnp.maximum(h + b1_ref[...], 0.0)
    h = jnp.dot(h, w2_ref[...], preferred_element_type=jnp.float32) + b2_ref[...]
    o_ref[...] = jnp.tanh(h)


def _mlp_call(x, w1, b1, w2, b2):
    return pl.pallas_call(
        _mlp_body,
        grid=(N_NODES // _BLK,),
        in_specs=[
            pl.BlockSpec((_BLK, IN_CH), lambda i: (i, 0)),
            pl.BlockSpec((IN_CH, HID), lambda i: (0, 0)),
            pl.BlockSpec((1, HID), lambda i: (0, 0)),
            pl.BlockSpec((HID, HID), lambda i: (0, 0)),
            pl.BlockSpec((1, HID), lambda i: (0, 0)),
        ],
        out_specs=pl.BlockSpec((_BLK, HID), lambda i: (i, 0)),
        out_shape=jax.ShapeDtypeStruct((N_NODES, HID), jnp.float32),
    )(x, w1, b1, w2, b2)


def _layer_body(a0_ref, a1_ref, a2_ref, a3_ref, h_ref, deg_ref,
                w0_ref, w1_ref, w2_ref, w3_ref, ws_ref, bg_ref,
                o_ref, *, relu):
    invd = 1.0 / jnp.maximum(deg_ref[:, :1], 1.0)
    acc = jnp.dot(a0_ref[...] * invd, w0_ref[...], preferred_element_type=jnp.float32)
    acc += jnp.dot(a1_ref[...] * invd, w1_ref[...], preferred_element_type=jnp.float32)
    acc += jnp.dot(a2_ref[...] * invd, w2_ref[...], preferred_element_type=jnp.float32)
    acc += jnp.dot(a3_ref[...] * invd, w3_ref[...], preferred_element_type=jnp.float32)
    acc += jnp.dot(h_ref[...], ws_ref[...], preferred_element_type=jnp.float32)
    acc += bg_ref[...]
    o_ref[...] = jnp.maximum(acc, 0.0) if relu else acc


def _layer_call(aq, h, deg, wagg, ws, bg, relu):
    return pl.pallas_call(
        functools.partial(_layer_body, relu=relu),
        grid=(N_NODES // _BLK,),
        in_specs=[
            pl.BlockSpec((_BLK, QTR), lambda i: (i, 0)),
            pl.BlockSpec((_BLK, QTR), lambda i: (i, 0)),
            pl.BlockSpec((_BLK, QTR), lambda i: (i, 0)),
            pl.BlockSpec((_BLK, QTR), lambda i: (i, 0)),
            pl.BlockSpec((_BLK, HID), lambda i: (i, 0)),
            pl.BlockSpec((_BLK, 16), lambda i: (i, 0)),
            pl.BlockSpec((QTR, HID), lambda i: (0, 0)),
            pl.BlockSpec((QTR, HID), lambda i: (0, 0)),
            pl.BlockSpec((QTR, HID), lambda i: (0, 0)),
            pl.BlockSpec((QTR, HID), lambda i: (0, 0)),
            pl.BlockSpec((HID, HID), lambda i: (0, 0)),
            pl.BlockSpec((1, HID), lambda i: (0, 0)),
        ],
        out_specs=pl.BlockSpec((_BLK, HID), lambda i: (i, 0)),
        out_shape=jax.ShapeDtypeStruct((N_NODES, HID), jnp.float32),
    )(aq[0], aq[1], aq[2], aq[3], h, deg,
      wagg[0 * QTR:1 * QTR], wagg[1 * QTR:2 * QTR],
      wagg[2 * QTR:3 * QTR], wagg[3 * QTR:4 * QTR], ws, bg)


def kernel(x, edge_index, W1, b1, W2, b2, Wagg, Wself, bg):
    src = edge_index[0].astype(jnp.int32)
    dst = edge_index[1].astype(jnp.int32)
    pad = E_PAD - N_EDGES
    src_p = jnp.concatenate([src, jnp.zeros((pad,), jnp.int32)])
    dst_p = jnp.concatenate(
        [dst, N_NODES + (jnp.arange(pad, dtype=jnp.int32) % N_DUMMY)]
    )
    q_off = 2 * jnp.arange(2, dtype=jnp.int32)[:, None, None] \
        + jnp.arange(2, dtype=jnp.int32)[None, :, None]
    gsrc = (4 * src_p)[None, None, :] + q_off
    gsrc = gsrc.reshape(2, 2, NUM_TILES, CHUNKS_PER_TILE, CHUNK)
    gdst = dst_p.reshape(NUM_TILES, CHUNKS_PER_TILE, CHUNK)

    zrow = jnp.zeros((CHUNK, QTR), jnp.float32)
    ones16 = jnp.ones((CHUNK, 16), jnp.float32)
    z16 = jnp.zeros((CHUNK, 16), jnp.float32)

    deg16 = _deg_call(gdst, ones16, z16)[0, :N_NODES]
    h = _mlp_call(x, W1, b1.reshape(1, HID), W2, b2.reshape(1, HID))
    for l in range(NUM_LAYERS):
        agg = _agg_call(h.reshape(4 * N_NODES, QTR), gsrc, gdst, zrow)
        aggq = agg.reshape(4, AGG_ROWS, QTR)[:, :N_NODES]
        h = _layer_call(aggq, h, deg16, Wagg[l], Wself[l],
                        bg[l].reshape(1, HID), relu=(l < NUM_LAYERS - 1))
    return h


# speedup vs baseline: 1.4701x; 1.4701x over previous
import functools

import jax
import jax.numpy as jnp
from jax import lax
from jax.experimental import pallas as pl
from jax.experimental.pallas import tpu as pltpu
from jax.experimental.pallas import tpu_sc as plsc

N_NODES = 10000
N_EDGES = 320000
IN_CH = 128
HID = 256
HALF = 128
QTR = 64
NUM_LAYERS = 3

NUM_TILES = 16
CHUNK = 128
BLOCK = 8
NUM_BLOCKS = 20
CHUNKS_PER_TILE = BLOCK * NUM_BLOCKS
E_PAD = NUM_TILES * CHUNKS_PER_TILE * CHUNK
AGG_ROWS = 10240
ROWS_PER_SUB = AGG_ROWS // NUM_TILES
TAB_PER_SUB = N_NODES // NUM_TILES
N_DUMMY = AGG_ROWS - N_NODES

_MESH = plsc.VectorSubcoreMesh(core_axis_name="c", subcore_axis_name="s")


def _agg_body(hq, gsrc, gdst, zrow, out, sring, dring, rows_a, rows_b,
              tab_sh, agg_sh, gsem_a, gsem_b, stsem):
    c = lax.axis_index("c")
    s = lax.axis_index("s")

    def gather(ring_h, ring_k, buf, sem):
        pltpu.async_copy(tab_sh.at[sring.at[ring_h, ring_k]], buf, sem)

    def gwait(buf, sem):
        pltpu.make_async_copy(tab_sh, buf, sem).wait()

    def scatter(ring_h, ring_k, buf):
        pltpu.sync_copy(buf, agg_sh.at[dring.at[ring_h, ring_k]], add=True)

    def stage_start(b, half):
        pltpu.async_copy(
            gsrc.at[s, pl.ds(b * BLOCK, BLOCK)], sring.at[half], stsem
        )
        pltpu.async_copy(
            gdst.at[s, pl.ds(b * BLOCK, BLOCK)], dring.at[half], stsem
        )

    def stage_wait(half):
        pltpu.make_async_copy(gsrc, sring.at[half], stsem).wait()
        pltpu.make_async_copy(gdst, dring.at[half], stsem).wait()

    for p in range(2):
        q = 2 * c + p
        pltpu.sync_copy(
            hq.at[q, pl.ds(s * TAB_PER_SUB, TAB_PER_SUB)],
            tab_sh.at[pl.ds(s * TAB_PER_SUB, TAB_PER_SUB)],
        )
        pltpu.sync_copy(zrow, rows_a)
        for t in range(ROWS_PER_SUB // CHUNK):
            pltpu.sync_copy(
                rows_a, agg_sh.at[pl.ds(s * ROWS_PER_SUB + t * CHUNK, CHUNK)]
            )
        plsc.subcore_barrier()

        stage_start(0, 0)
        stage_wait(0)
        gather(0, 0, rows_a, gsem_a)

        def block_step(b, carry):
            half = lax.rem(b, 2)
            nxt = lax.rem(b + 1, 2)

            @pl.when(b < NUM_BLOCKS - 1)
            def _():
                stage_start(b + 1, nxt)

            for k in range(BLOCK):
                cur, csem = (rows_a, gsem_a) if k % 2 == 0 else (rows_b, gsem_b)
                oth, osem = (rows_b, gsem_b) if k % 2 == 0 else (rows_a, gsem_a)
                if k < BLOCK - 1:
                    gather(half, k + 1, oth, osem)
                else:
                    @pl.when(b < NUM_BLOCKS - 1)
                    def _():
                        stage_wait(nxt)
                        gather(nxt, 0, oth, osem)
                gwait(cur, csem)
                scatter(half, k, cur)
            return carry

        lax.fori_loop(0, NUM_BLOCKS, block_step, 0)
        plsc.subcore_barrier()
        pltpu.sync_copy(
            agg_sh.at[pl.ds(s * ROWS_PER_SUB, ROWS_PER_SUB)],
            out.at[c, p, pl.ds(s * ROWS_PER_SUB, ROWS_PER_SUB)],
        )


_agg_call = pl.kernel(
    _agg_body,
    out_type=jax.ShapeDtypeStruct((2, 2, AGG_ROWS, QTR), jnp.float32),
    mesh=_MESH,
    compiler_params=pltpu.CompilerParams(use_tc_tiling_on_sc=False),
    scratch_types=[
        pltpu.VMEM((2, BLOCK, CHUNK), jnp.int32),
        pltpu.VMEM((2, BLOCK, CHUNK), jnp.int32),
        pltpu.VMEM((CHUNK, QTR), jnp.float32),
        pltpu.VMEM((CHUNK, QTR), jnp.float32),
        pltpu.VMEM_SHARED((N_NODES, QTR), jnp.float32),
        pltpu.VMEM_SHARED((AGG_ROWS, QTR), jnp.float32),
        pltpu.SemaphoreType.DMA,
        pltpu.SemaphoreType.DMA,
        pltpu.SemaphoreType.DMA,
    ],
)


def _deg_body(gdst, ones_v_src, z_v_src, out, gdst_v, ones_v, z_v, deg_sh):
    c = lax.axis_index("c")
    s = lax.axis_index("s")
    pltpu.sync_copy(gdst.at[s], gdst_v)
    pltpu.sync_copy(ones_v_src, ones_v)
    pltpu.sync_copy(z_v_src, z_v)
    for t in range(ROWS_PER_SUB // CHUNK):
        pltpu.sync_copy(z_v, deg_sh.at[pl.ds(s * ROWS_PER_SUB + t * CHUNK, CHUNK)])
    plsc.subcore_barrier()

    def chunk_step(j, carry):
        pltpu.sync_copy(ones_v, deg_sh.at[gdst_v.at[j]], add=True)
        return carry

    lax.fori_loop(0, CHUNKS_PER_TILE, chunk_step, 0)
    plsc.subcore_barrier()
    pltpu.sync_copy(
        deg_sh.at[pl.ds(s * ROWS_PER_SUB, ROWS_PER_SUB)],
        out.at[c, pl.ds(s * ROWS_PER_SUB, ROWS_PER_SUB)],
    )


_deg_call = pl.kernel(
    _deg_body,
    out_type=jax.ShapeDtypeStruct((2, AGG_ROWS, 16), jnp.float32),
    mesh=_MESH,
    compiler_params=pltpu.CompilerParams(use_tc_tiling_on_sc=False),
    scratch_types=[
        pltpu.VMEM((CHUNKS_PER_TILE, CHUNK), jnp.int32),
        pltpu.VMEM((CHUNK, 16), jnp.float32),
        pltpu.VMEM((CHUNK, 16), jnp.float32),
        pltpu.VMEM_SHARED((AGG_ROWS, 16), jnp.float32),
    ],
)


_BLK = 2000


def _write_quarters(oq_ref, acc):
    for qq in range(4):
        oq_ref[qq] = acc[:, qq * QTR:(qq + 1) * QTR]


def _mlp_body(x_ref, w1_ref, b1_ref, w2_ref, b2_ref, o_ref, oq_ref):
    h = jnp.dot(x_ref[...], w1_ref[...], preferred_element_type=jnp.float32)
    h = jnp.maximum(h + b1_ref[...], 0.0)
    h = jnp.dot(h, w2_ref[...], preferred_element_type=jnp.float32) + b2_ref[...]
    h = jnp.tanh(h)
    o_ref[...] = h
    _write_quarters(oq_ref, h)


def _mlp_call(x, w1, b1, w2, b2):
    return pl.pallas_call(
        _mlp_body,
        grid=(N_NODES // _BLK,),
        in_specs=[
            pl.BlockSpec((_BLK, IN_CH), lambda i: (i, 0)),
            pl.BlockSpec((IN_CH, HID), lambda i: (0, 0)),
            pl.BlockSpec((1, HID), lambda i: (0, 0)),
            pl.BlockSpec((HID, HID), lambda i: (0, 0)),
            pl.BlockSpec((1, HID), lambda i: (0, 0)),
        ],
        out_specs=[
            pl.BlockSpec((_BLK, HID), lambda i: (i, 0)),
            pl.BlockSpec((4, _BLK, QTR), lambda i: (0, i, 0)),
        ],
        out_shape=[
            jax.ShapeDtypeStruct((N_NODES, HID), jnp.float32),
            jax.ShapeDtypeStruct((4, N_NODES, QTR), jnp.float32),
        ],
    )(x, w1, b1, w2, b2)


def _layer_body(a0_ref, a1_ref, a2_ref, a3_ref, h_ref, deg_ref,
                w0_ref, w1_ref, w2_ref, w3_ref, ws_ref, bg_ref,
                o_ref, oq_ref, *, relu):
    invd = 1.0 / jnp.maximum(deg_ref[:, :1], 1.0)
    acc = jnp.dot(a0_ref[...] * invd, w0_ref[...], preferred_element_type=jnp.float32)
    acc += jnp.dot(a1_ref[...] * invd, w1_ref[...], preferred_element_type=jnp.float32)
    acc += jnp.dot(a2_ref[...] * invd, w2_ref[...], preferred_element_type=jnp.float32)
    acc += jnp.dot(a3_ref[...] * invd, w3_ref[...], preferred_element_type=jnp.float32)
    acc += jnp.dot(h_ref[...], ws_ref[...], preferred_element_type=jnp.float32)
    acc += bg_ref[...]
    if relu:
        acc = jnp.maximum(acc, 0.0)
    o_ref[...] = acc
    _write_quarters(oq_ref, acc)


def _layer_call(aq, h, deg, wagg, ws, bg, relu):
    return pl.pallas_call(
        functools.partial(_layer_body, relu=relu),
        grid=(N_NODES // _BLK,),
        in_specs=[
            pl.BlockSpec((_BLK, QTR), lambda i: (i, 0)),
            pl.BlockSpec((_BLK, QTR), lambda i: (i, 0)),
            pl.BlockSpec((_BLK, QTR), lambda i: (i, 0)),
            pl.BlockSpec((_BLK, QTR), lambda i: (i, 0)),
            pl.BlockSpec((_BLK, HID), lambda i: (i, 0)),
            pl.BlockSpec((_BLK, 16), lambda i: (i, 0)),
            pl.BlockSpec((QTR, HID), lambda i: (0, 0)),
            pl.BlockSpec((QTR, HID), lambda i: (0, 0)),
            pl.BlockSpec((QTR, HID), lambda i: (0, 0)),
            pl.BlockSpec((QTR, HID), lambda i: (0, 0)),
            pl.BlockSpec((HID, HID), lambda i: (0, 0)),
            pl.BlockSpec((1, HID), lambda i: (0, 0)),
        ],
        out_specs=[
            pl.BlockSpec((_BLK, HID), lambda i: (i, 0)),
            pl.BlockSpec((4, _BLK, QTR), lambda i: (0, i, 0)),
        ],
        out_shape=[
            jax.ShapeDtypeStruct((N_NODES, HID), jnp.float32),
            jax.ShapeDtypeStruct((4, N_NODES, QTR), jnp.float32),
        ],
    )(aq[0], aq[1], aq[2], aq[3], h, deg,
      wagg[0 * QTR:1 * QTR], wagg[1 * QTR:2 * QTR],
      wagg[2 * QTR:3 * QTR], wagg[3 * QTR:4 * QTR], ws, bg)


def kernel(x, edge_index, W1, b1, W2, b2, Wagg, Wself, bg):
    src = edge_index[0].astype(jnp.int32)
    dst = edge_index[1].astype(jnp.int32)
    pad = E_PAD - N_EDGES
    src_p = jnp.concatenate([src, jnp.zeros((pad,), jnp.int32)])
    dst_p = jnp.concatenate(
        [dst, N_NODES + (jnp.arange(pad, dtype=jnp.int32) % N_DUMMY)]
    )
    gsrc = src_p.reshape(NUM_TILES, CHUNKS_PER_TILE, CHUNK)
    gdst = dst_p.reshape(NUM_TILES, CHUNKS_PER_TILE, CHUNK)

    zrow = jnp.zeros((CHUNK, QTR), jnp.float32)
    ones16 = jnp.ones((CHUNK, 16), jnp.float32)
    z16 = jnp.zeros((CHUNK, 16), jnp.float32)

    deg16 = _deg_call(gdst, ones16, z16)[0, :N_NODES]
    h, hq = _mlp_call(x, W1, b1.reshape(1, HID), W2, b2.reshape(1, HID))
    for l in range(NUM_LAYERS):
        agg = _agg_call(hq, gsrc, gdst, zrow)
        aggq = agg.reshape(4, AGG_ROWS, QTR)[:, :N_NODES]
        h, hq = _layer_call(aggq, h, deg16, Wagg[l], Wself[l],
                            bg[l].reshape(1, HID), relu=(l < NUM_LAYERS - 1))
    return h
